# baseline (speedup 1.0000x reference)
import functools

import jax
import jax.numpy as jnp
from jax import lax
from jax.experimental import pallas as pl
from jax.experimental.pallas import tpu as pltpu, tpu_sc as plsc

N = 32768
NRAYS = 16
NC = 2
NSUB = 16
CH = N // NSUB
SH = 11
L = 16
NV = CH // L
EPS = 1e-6
_LN2 = 0.6931471805599453
_SQRT2 = 1.4142135623730951

_mesh = plsc.VectorSubcoreMesh(
    core_axis_name="c", subcore_axis_name="s", num_cores=NC, num_subcores=NSUB
)


def _ln16(x):
    bits = lax.bitcast_convert_type(x, jnp.int32)
    e = lax.shift_right_arithmetic(bits, 23) - 127
    m = lax.bitcast_convert_type(
        (bits & 0x007FFFFF) | 0x3F800000, jnp.float32
    )
    big = m > _SQRT2
    m = jnp.where(big, m * 0.5, m)
    e = jnp.where(big, e + 1, e)
    s = (m - 1.0) / (m + 1.0)
    z = s * s
    p = 1.0 + z * (
        (1.0 / 3.0)
        + z * ((1.0 / 5.0) + z * ((1.0 / 7.0) + z * ((1.0 / 9.0) + z * (1.0 / 11.0))))
    )
    return 2.0 * s * p + e.astype(jnp.float32) * _LN2


def _body(
    cu_hbm, alpha_hbm, trans_hbm, bg_hbm, summ_hbm,
    alpha_v, ex_v, cu_v, pub_v, all_v, ci_v, r_v, bg_v,
):
    s = lax.axis_index("s")
    c = lax.axis_index("c")
    base = s * CH
    pltpu.sync_copy(alpha_hbm.at[pl.ds(base, CH)], alpha_v)
    pltpu.sync_copy(cu_hbm.at[pl.ds(0, L)], cu_v)

    def body1(v, carry):
        a = alpha_v[pl.ds(v * L, L)]
        x = jnp.minimum(jnp.maximum(1.0 - a, EPS), 1.0)
        l = _ln16(x)
        cv = plsc.cumsum(l)
        ex_v[pl.ds(v * L, L)] = (cv - l) + carry
        return carry + cv[L - 1]

    total = lax.fori_loop(0, NV, body1, jnp.float32(0.0))

    S = cu_v[...]
    in_s = (S >= base) & (S < base + CH)
    q = jnp.clip(S - base, 0, CH - 1)
    loc = plsc.load_gather(ex_v, [q])
    pub_v[0, :] = jnp.where(in_s, loc, 0.0)
    pub_v[1, :] = jnp.full((L,), total, jnp.float32)
    pltpu.sync_copy(pub_v, summ_hbm.at[s])
    plsc.subcore_barrier()
    pltpu.sync_copy(summ_hbm, all_v)

    iot = lax.iota(jnp.int32, L)
    sv = jnp.zeros((L,), jnp.float32)
    for w in range(NSUB):
        sv = sv + all_v[w, 0, :]

    c_sv = lax.shift_right_arithmetic(S, SH)
    idx1 = jnp.minimum(iot + 1, L - 1)
    last = iot == L - 1
    ci_v[...] = c_sv
    c_ev = jnp.where(last, NSUB, plsc.load_gather(ci_v, [idx1]))
    r_v[...] = sv
    ev = jnp.where(last, 0.0, plsc.load_gather(r_v, [idx1]))

    D = jnp.zeros((L,), jnp.float32)
    BD = jnp.zeros((L,), jnp.float32)
    for w in range(NSUB):
        tot_w = all_v[w, 1, :][0]
        m_ge = c_sv <= w
        D = D + jnp.where(m_ge & (w < s), tot_w, 0.0)
        BD = BD + jnp.where(m_ge & (w < c_ev), tot_w, 0.0)

    bg_v[...] = jnp.exp((ev - sv) + BD)

    @pl.when((s == 0) & (c == 0))
    def _():
        pltpu.sync_copy(bg_v, bg_hbm)

    rvec = D - sv
    rs = [rvec[j] for j in range(L)]
    cus = [S[j] for j in range(1, L)]

    def body2(v, carry):
        ex = ex_v[pl.ds(v * L, L)]
        p = (base + v * L) + iot
        acc = jnp.full((L,), rs[0], jnp.float32)
        for j in range(1, L):
            acc = jnp.where(p >= cus[j - 1], rs[j], acc)
        ex_v[pl.ds(v * L, L)] = jnp.exp(ex + acc)
        return carry

    lax.fori_loop(0, NV, body2, jnp.int32(0))

    @pl.when(c == 0)
    def _():
        pltpu.sync_copy(ex_v, trans_hbm.at[pl.ds(base, CH)])


_kernel = functools.partial(
    pl.kernel,
    out_type=(
        jax.ShapeDtypeStruct((N,), jnp.float32),
        jax.ShapeDtypeStruct((NRAYS,), jnp.float32),
        jax.ShapeDtypeStruct((NSUB, 2, L), jnp.float32),
    ),
    mesh=_mesh,
    scratch_types=[
        pltpu.VMEM((CH,), jnp.float32),
        pltpu.VMEM((CH,), jnp.float32),
        pltpu.VMEM((L,), jnp.int32),
        pltpu.VMEM((2, L), jnp.float32),
        pltpu.VMEM((NSUB, 2, L), jnp.float32),
        pltpu.VMEM((L,), jnp.int32),
        pltpu.VMEM((L,), jnp.float32),
        pltpu.VMEM((L,), jnp.float32),
    ],
    compiler_params=pltpu.CompilerParams(needs_layout_passes=False),
)(_body)


def kernel(cu_seqlens, alpha):
    transmittance, bg_transmittance, _ = _kernel(cu_seqlens, alpha)
    return transmittance, bg_transmittance
